# Initial kernel scaffold; baseline (speedup 1.0000x reference)
import jax
import jax.numpy as jnp
from jax.experimental import pallas as pl


def kernel(x, edge_index, W1, b1, g1, be1, W2, b2, g2, be2, W3, b3):
    raise NotImplementedError("write your pallas kernel here")


# speedup vs baseline: 16.7910x; 16.7910x over previous
import functools

import jax
import jax.numpy as jnp
from jax import lax
from jax.experimental import pallas as pl
from jax.experimental.pallas import tpu as pltpu
import jax.experimental.pallas.tpu_sc as plsc

NC = 2
NS = 16
NW = NC * NS
LANES = 16


def _mesh():
    return plsc.VectorSubcoreMesh(
        core_axis_name="c", subcore_axis_name="s",
        num_cores=NC, num_subcores=NS)


_SC_PARAMS = pltpu.CompilerParams(needs_layout_passes=False)


@functools.cache
def _make_count(n, hw, ng, g):
    ra = (((n + NS - 1) // NS + 7) // 8) * 8
    rb = n - (NS - 1) * ra

    def body(dst_hbm, ones_hbm, zero_hbm, out_hbm, dst_v, ones_v, acc_sh):
        c = lax.axis_index("c")
        s = lax.axis_index("s")
        wid = c * NS + s
        pltpu.sync_copy(dst_hbm.at[wid], dst_v)
        pltpu.sync_copy(ones_hbm, ones_v)
        base = s * ra

        @pl.when(s < NS - 1)
        def _():
            pltpu.sync_copy(zero_hbm, acc_sh.at[pl.ds(base, ra)])

        @pl.when(s == NS - 1)
        def _():
            pltpu.sync_copy(zero_hbm.at[pl.ds(0, rb)],
                            acc_sh.at[pl.ds((NS - 1) * ra, rb)])
        plsc.subcore_barrier()

        def grp(j, _):
            pltpu.sync_copy(ones_v, acc_sh.at[dst_v.at[j]], add=True)
            return 0
        lax.fori_loop(0, ng, grp, 0)

        plsc.subcore_barrier()

        @pl.when(s < NS - 1)
        def _():
            pltpu.sync_copy(acc_sh.at[pl.ds(base, ra)],
                            out_hbm.at[c, pl.ds(base, ra)])

        @pl.when(s == NS - 1)
        def _():
            pltpu.sync_copy(acc_sh.at[pl.ds((NS - 1) * ra, rb)],
                            out_hbm.at[c, pl.ds((NS - 1) * ra, rb)])

    return pl.kernel(
        body,
        out_type=jax.ShapeDtypeStruct((NC, n, hw), jnp.float32),
        mesh=_mesh(),
        compiler_params=_SC_PARAMS,
        scratch_types=[
            pltpu.VMEM((ng, g), jnp.int32),
            pltpu.VMEM((g, hw), jnp.float32),
            pltpu.VMEM_SHARED((n, hw), jnp.float32),
        ],
    )


@functools.cache
def _make_scatter(n, hw, ng, g):
    ra = (((n + NS - 1) // NS + 7) // 8) * 8
    rb = n - (NS - 1) * ra
    assert rb > 0 and rb % 8 == 0

    def body(y_hbm, src_hbm, dst_hbm, zero_hbm, out_hbm,
             src_v, dst_v, rows_v, acc_sh, sem):
        c = lax.axis_index("c")
        s = lax.axis_index("s")
        wid = c * NS + s
        pltpu.sync_copy(src_hbm.at[wid], src_v)
        pltpu.sync_copy(dst_hbm.at[wid], dst_v)

        base = s * ra

        @pl.when(s < NS - 1)
        def _():
            pltpu.sync_copy(zero_hbm, acc_sh.at[pl.ds(base, ra)])

        @pl.when(s == NS - 1)
        def _():
            pltpu.sync_copy(zero_hbm.at[pl.ds(0, rb)],
                            acc_sh.at[pl.ds((NS - 1) * ra, rb)])
        plsc.subcore_barrier()

        def grp(j, _):
            pltpu.async_copy(y_hbm.at[src_v.at[j]], rows_v, sem).wait()
            pltpu.sync_copy(rows_v, acc_sh.at[dst_v.at[j]], add=True)
            return 0
        lax.fori_loop(0, ng, grp, 0)

        plsc.subcore_barrier()

        @pl.when(s < NS - 1)
        def _():
            pltpu.sync_copy(acc_sh.at[pl.ds(base, ra)],
                            out_hbm.at[c, pl.ds(base, ra)])

        @pl.when(s == NS - 1)
        def _():
            pltpu.sync_copy(acc_sh.at[pl.ds((NS - 1) * ra, rb)],
                            out_hbm.at[c, pl.ds((NS - 1) * ra, rb)])

    return pl.kernel(
        body,
        out_type=jax.ShapeDtypeStruct((NC, n, hw), jnp.float32),
        mesh=_mesh(),
        compiler_params=_SC_PARAMS,
        scratch_types=[
            pltpu.VMEM((ng, g), jnp.int32),
            pltpu.VMEM((ng, g), jnp.int32),
            pltpu.VMEM((g, hw), jnp.float32),
            pltpu.VMEM_SHARED((n, hw), jnp.float32),
            pltpu.SemaphoreType.DMA,
        ],
    )


@functools.cache
def _make_dinv(n):
    def body(deg_ref, out_ref):
        d = deg_ref[0:1, :] + deg_ref[1:2, :] + 1.0
        out_ref[...] = lax.rsqrt(d)

    return pl.pallas_call(
        body, out_shape=jax.ShapeDtypeStruct((1, n), jnp.float32))


@functools.cache
def _make_lin(n, d, h, br):
    def body(x_ref, w_ref, dinv_ref, out_ref):
        xw = jnp.dot(x_ref[...], w_ref[...],
                     preferred_element_type=jnp.float32)
        out_ref[...] = xw * dinv_ref[...]

    return pl.pallas_call(
        body,
        grid=(n // br,),
        in_specs=[
            pl.BlockSpec((br, d), lambda i: (i, 0)),
            pl.BlockSpec((d, h), lambda i: (0, 0)),
            pl.BlockSpec((br, 1), lambda i: (i, 0)),
        ],
        out_specs=pl.BlockSpec((br, h), lambda i: (i, 0)),
        out_shape=jax.ShapeDtypeStruct((n, h), jnp.float32),
    )


@functools.cache
def _make_post(n, hw, br):
    def body(p_ref, y_ref, dinv_ref, b_ref, z_ref, s1_ref, s2_ref):
        i = pl.program_id(0)
        z = dinv_ref[...] * (p_ref[0] + p_ref[1] + y_ref[...]) + b_ref[...]
        z_ref[...] = z

        @pl.when(i == 0)
        def _():
            s1_ref[...] = jnp.zeros_like(s1_ref)
            s2_ref[...] = jnp.zeros_like(s2_ref)
        s1_ref[...] += jnp.sum(z, axis=0, keepdims=True)
        s2_ref[...] += jnp.sum(z * z, axis=0, keepdims=True)

    return pl.pallas_call(
        body,
        grid=(n // br,),
        in_specs=[
            pl.BlockSpec((NC, br, hw), lambda i: (0, i, 0)),
            pl.BlockSpec((br, hw), lambda i: (i, 0)),
            pl.BlockSpec((br, 1), lambda i: (i, 0)),
            pl.BlockSpec((1, hw), lambda i: (0, 0)),
        ],
        out_specs=[
            pl.BlockSpec((br, hw), lambda i: (i, 0)),
            pl.BlockSpec((1, hw), lambda i: (0, 0)),
            pl.BlockSpec((1, hw), lambda i: (0, 0)),
        ],
        out_shape=[
            jax.ShapeDtypeStruct((n, hw), jnp.float32),
            jax.ShapeDtypeStruct((1, hw), jnp.float32),
            jax.ShapeDtypeStruct((1, hw), jnp.float32),
        ],
    )


@functools.cache
def _make_bnmm(n, hw, hout, br):
    inv_n = 1.0 / n

    def body(z_ref, s1_ref, s2_ref, g_ref, be_ref, w_ref, dinv_ref, out_ref):
        m = s1_ref[...] * inv_n
        var = s2_ref[...] * inv_n - m * m
        istd = lax.rsqrt(var + 1e-5)
        h = jnp.maximum((z_ref[...] - m) * (istd * g_ref[...]) + be_ref[...],
                        0.0)
        out_ref[...] = jnp.dot(h, w_ref[...],
                               preferred_element_type=jnp.float32) * dinv_ref[...]

    return pl.pallas_call(
        body,
        grid=(n // br,),
        in_specs=[
            pl.BlockSpec((br, hw), lambda i: (i, 0)),
            pl.BlockSpec((1, hw), lambda i: (0, 0)),
            pl.BlockSpec((1, hw), lambda i: (0, 0)),
            pl.BlockSpec((1, hw), lambda i: (0, 0)),
            pl.BlockSpec((1, hw), lambda i: (0, 0)),
            pl.BlockSpec((hw, hout), lambda i: (0, 0)),
            pl.BlockSpec((br, 1), lambda i: (i, 0)),
        ],
        out_specs=pl.BlockSpec((br, hout), lambda i: (i, 0)),
        out_shape=jax.ShapeDtypeStruct((n, hout), jnp.float32),
    )


@functools.cache
def _make_comb(n, hw, br):
    def body(p_ref, y_ref, dinv_ref, b_ref, out_ref):
        out_ref[...] = (dinv_ref[...] * (p_ref[0] + p_ref[1] + y_ref[...])
                        + b_ref[...])

    return pl.pallas_call(
        body,
        grid=(n // br,),
        in_specs=[
            pl.BlockSpec((NC, br, hw), lambda i: (0, i, 0)),
            pl.BlockSpec((br, hw), lambda i: (i, 0)),
            pl.BlockSpec((br, 1), lambda i: (i, 0)),
            pl.BlockSpec((1, hw), lambda i: (0, 0)),
        ],
        out_specs=pl.BlockSpec((br, hw), lambda i: (i, 0)),
        out_shape=jax.ShapeDtypeStruct((n, hw), jnp.float32),
    )


def kernel(x, edge_index, W1, b1, g1, be1, W2, b2, g2, be2, W3, b3):
    n, d = x.shape
    h = W1.shape[1]
    c_out = W3.shape[1]
    e = edge_index.shape[1]
    epw = e // NW
    g = 125
    ng = epw // g

    src3 = edge_index[0].reshape(NW, ng, g)
    dst3 = edge_index[1].reshape(NW, ng, g)

    br = 1000
    ra = (((n + NS - 1) // NS + 7) // 8) * 8
    zeros_h = jnp.zeros((ra, h), jnp.float32)
    ones_h = jnp.ones((g, h), jnp.float32)

    cnt = _make_count(n, h, ng, g)(dst3, ones_h, zeros_h)
    deg2 = cnt[:, :, 0]
    dinv_col = _make_dinv(n)(deg2).reshape(n, 1)

    y1 = _make_lin(n, d, h, br)(x, W1, dinv_col)
    p1 = _make_scatter(n, h, ng, g)(y1, src3, dst3, zeros_h)
    z1, s11, s12 = _make_post(n, h, br)(p1, y1, dinv_col, b1.reshape(1, h))
    y2 = _make_bnmm(n, h, h, br)(z1, s11, s12, g1.reshape(1, h),
                                 be1.reshape(1, h), W2, dinv_col)
    p2 = _make_scatter(n, h, ng, g)(y2, src3, dst3, zeros_h)
    z2, s21, s22 = _make_post(n, h, br)(p2, y2, dinv_col, b2.reshape(1, h))

    cp = h
    W3p = jnp.pad(W3, ((0, 0), (0, cp - c_out)))
    b3p = jnp.pad(b3, (0, cp - c_out)).reshape(1, cp)
    y3 = _make_bnmm(n, h, cp, br)(z2, s21, s22, g2.reshape(1, h),
                                  be2.reshape(1, h), W3p, dinv_col)
    zeros_c = jnp.zeros((ra, cp), jnp.float32)
    p3 = _make_scatter(n, cp, ng, g)(y3, src3, dst3, zeros_c)
    out = _make_comb(n, cp, br)(p3, y3, dinv_col, b3p)
    return out[:, :c_out]


# speedup vs baseline: 22.0572x; 1.3136x over previous
import functools

import jax
import jax.numpy as jnp
from jax import lax
from jax.experimental import pallas as pl
from jax.experimental.pallas import tpu as pltpu
import jax.experimental.pallas.tpu_sc as plsc

NC = 2
NS = 16
NW = NC * NS
LANES = 16


def _mesh():
    return plsc.VectorSubcoreMesh(
        core_axis_name="c", subcore_axis_name="s",
        num_cores=NC, num_subcores=NS)


_SC_PARAMS = pltpu.CompilerParams(needs_layout_passes=False)


@functools.cache
def _make_count(n, hw, ng, g):
    ra = (((n + NS - 1) // NS + 7) // 8) * 8
    rb = n - (NS - 1) * ra

    def body(dst_hbm, ones_hbm, zero_hbm, out_hbm, dst_v, ones_v, acc_sh,
             sem):
        c = lax.axis_index("c")
        s = lax.axis_index("s")
        wid = c * NS + s
        pltpu.sync_copy(dst_hbm.at[wid], dst_v)
        pltpu.sync_copy(ones_hbm, ones_v)
        base = s * ra

        @pl.when(s < NS - 1)
        def _():
            pltpu.sync_copy(zero_hbm, acc_sh.at[pl.ds(base, ra)])

        @pl.when(s == NS - 1)
        def _():
            pltpu.sync_copy(zero_hbm.at[pl.ds(0, rb)],
                            acc_sh.at[pl.ds((NS - 1) * ra, rb)])
        plsc.subcore_barrier()

        def grp(j, _):
            pltpu.sync_copy(ones_v, acc_sh.at[dst_v.at[j]], add=True)
            return 0
        lax.fori_loop(0, ng, grp, 0)

        plsc.subcore_barrier()

        @pl.when(s < NS - 1)
        def _():
            pltpu.sync_copy(acc_sh.at[pl.ds(base, ra)],
                            out_hbm.at[c, pl.ds(base, ra)])

        @pl.when(s == NS - 1)
        def _():
            pltpu.sync_copy(acc_sh.at[pl.ds((NS - 1) * ra, rb)],
                            out_hbm.at[c, pl.ds((NS - 1) * ra, rb)])

    return pl.kernel(
        body,
        out_type=jax.ShapeDtypeStruct((NC, n, hw), jnp.float32),
        mesh=_mesh(),
        compiler_params=_SC_PARAMS,
        scratch_types=[
            pltpu.VMEM((ng, g), jnp.int32),
            pltpu.VMEM((g, hw), jnp.float32),
            pltpu.VMEM_SHARED((n, hw), jnp.float32),
            pltpu.SemaphoreType.DMA,
        ],
    )


@functools.cache
def _make_scatter(n, hw, ng, g):
    ra = (((n + NS - 1) // NS + 7) // 8) * 8
    rb = n - (NS - 1) * ra
    assert rb > 0 and rb % 8 == 0

    epw = ng * g

    def body(y_hbm, src_hbm, dst_hbm, zero_hbm, out_hbm,
             src_v, dst_v, rows_v, acc_sh, sem0, sem1):
        c = lax.axis_index("c")
        s = lax.axis_index("s")
        wid = c * NS + s
        pltpu.sync_copy(src_hbm.at[wid], src_v)
        pltpu.sync_copy(dst_hbm.at[wid], dst_v)
        rows0_v = rows_v.at[0]
        rows1_v = rows_v.at[1]

        base = s * ra

        @pl.when(s < NS - 1)
        def _():
            pltpu.sync_copy(zero_hbm, acc_sh.at[pl.ds(base, ra)])

        @pl.when(s == NS - 1)
        def _():
            pltpu.sync_copy(zero_hbm.at[pl.ds(0, rb)],
                            acc_sh.at[pl.ds((NS - 1) * ra, rb)])
        plsc.subcore_barrier()

        def src_at(j):
            return src_v.at[pl.ds(pl.multiple_of(j * g, 8), g)]

        pltpu.async_copy(y_hbm.at[src_at(0)], rows0_v, sem0)
        pltpu.async_copy(y_hbm.at[src_at(1)], rows1_v, sem1)

        def grp2(j2, _):
            for b, (buf, sem) in enumerate(((rows0_v, sem0),
                                            (rows1_v, sem1))):
                j = j2 * 2 + b
                pltpu.make_async_copy(y_hbm.at[src_at(j)], buf, sem).wait()
                pltpu.sync_copy(buf, acc_sh.at[dst_v.at[j]], add=True)

                @pl.when(j + 2 < ng)
                def _():
                    pltpu.async_copy(y_hbm.at[src_at(j + 2)], buf, sem)
            return 0
        lax.fori_loop(0, ng // 2, grp2, 0)

        if ng % 2:
            pltpu.make_async_copy(y_hbm.at[src_at(ng - 1)], rows0_v,
                                  sem0).wait()
            pltpu.sync_copy(rows0_v, acc_sh.at[dst_v.at[ng - 1]], add=True)

        plsc.subcore_barrier()

        @pl.when(s < NS - 1)
        def _():
            pltpu.sync_copy(acc_sh.at[pl.ds(base, ra)],
                            out_hbm.at[c, pl.ds(base, ra)])

        @pl.when(s == NS - 1)
        def _():
            pltpu.sync_copy(acc_sh.at[pl.ds((NS - 1) * ra, rb)],
                            out_hbm.at[c, pl.ds((NS - 1) * ra, rb)])

    return pl.kernel(
        body,
        out_type=jax.ShapeDtypeStruct((NC, n, hw), jnp.float32),
        mesh=_mesh(),
        compiler_params=_SC_PARAMS,
        scratch_types=[
            pltpu.VMEM((epw,), jnp.int32),
            pltpu.VMEM((ng, g), jnp.int32),
            pltpu.VMEM((2, g, hw), jnp.float32),
            pltpu.VMEM_SHARED((n, hw), jnp.float32),
            pltpu.SemaphoreType.DMA,
            pltpu.SemaphoreType.DMA,
        ],
    )


@functools.cache
def _make_dinv(n):
    def body(deg_ref, out_ref):
        d = deg_ref[0:1, :] + deg_ref[1:2, :] + 1.0
        out_ref[...] = lax.rsqrt(d)

    return pl.pallas_call(
        body, out_shape=jax.ShapeDtypeStruct((1, n), jnp.float32))


@functools.cache
def _make_lin(n, d, h, br):
    def body(x_ref, w_ref, dinv_ref, out_ref):
        xw = jnp.dot(x_ref[...], w_ref[...],
                     preferred_element_type=jnp.float32)
        out_ref[...] = xw * dinv_ref[...]

    return pl.pallas_call(
        body,
        grid=(n // br,),
        in_specs=[
            pl.BlockSpec((br, d), lambda i: (i, 0)),
            pl.BlockSpec((d, h), lambda i: (0, 0)),
            pl.BlockSpec((br, 1), lambda i: (i, 0)),
        ],
        out_specs=pl.BlockSpec((br, h), lambda i: (i, 0)),
        out_shape=jax.ShapeDtypeStruct((n, h), jnp.float32),
    )


@functools.cache
def _make_post(n, hw, br):
    def body(p_ref, y_ref, dinv_ref, b_ref, z_ref, s1_ref, s2_ref):
        i = pl.program_id(0)
        z = dinv_ref[...] * (p_ref[0] + p_ref[1] + y_ref[...]) + b_ref[...]
        z_ref[...] = z

        @pl.when(i == 0)
        def _():
            s1_ref[...] = jnp.zeros_like(s1_ref)
            s2_ref[...] = jnp.zeros_like(s2_ref)
        s1_ref[...] += jnp.sum(z, axis=0, keepdims=True)
        s2_ref[...] += jnp.sum(z * z, axis=0, keepdims=True)

    return pl.pallas_call(
        body,
        grid=(n // br,),
        in_specs=[
            pl.BlockSpec((NC, br, hw), lambda i: (0, i, 0)),
            pl.BlockSpec((br, hw), lambda i: (i, 0)),
            pl.BlockSpec((br, 1), lambda i: (i, 0)),
            pl.BlockSpec((1, hw), lambda i: (0, 0)),
        ],
        out_specs=[
            pl.BlockSpec((br, hw), lambda i: (i, 0)),
            pl.BlockSpec((1, hw), lambda i: (0, 0)),
            pl.BlockSpec((1, hw), lambda i: (0, 0)),
        ],
        out_shape=[
            jax.ShapeDtypeStruct((n, hw), jnp.float32),
            jax.ShapeDtypeStruct((1, hw), jnp.float32),
            jax.ShapeDtypeStruct((1, hw), jnp.float32),
        ],
    )


@functools.cache
def _make_bnmm(n, hw, hout, br):
    inv_n = 1.0 / n

    def body(z_ref, s1_ref, s2_ref, g_ref, be_ref, w_ref, dinv_ref, out_ref):
        m = s1_ref[...] * inv_n
        var = s2_ref[...] * inv_n - m * m
        istd = lax.rsqrt(var + 1e-5)
        h = jnp.maximum((z_ref[...] - m) * (istd * g_ref[...]) + be_ref[...],
                        0.0)
        out_ref[...] = jnp.dot(h, w_ref[...],
                               preferred_element_type=jnp.float32) * dinv_ref[...]

    return pl.pallas_call(
        body,
        grid=(n // br,),
        in_specs=[
            pl.BlockSpec((br, hw), lambda i: (i, 0)),
            pl.BlockSpec((1, hw), lambda i: (0, 0)),
            pl.BlockSpec((1, hw), lambda i: (0, 0)),
            pl.BlockSpec((1, hw), lambda i: (0, 0)),
            pl.BlockSpec((1, hw), lambda i: (0, 0)),
            pl.BlockSpec((hw, hout), lambda i: (0, 0)),
            pl.BlockSpec((br, 1), lambda i: (i, 0)),
        ],
        out_specs=pl.BlockSpec((br, hout), lambda i: (i, 0)),
        out_shape=jax.ShapeDtypeStruct((n, hout), jnp.float32),
    )


@functools.cache
def _make_comb(n, hw, br):
    def body(p_ref, y_ref, dinv_ref, b_ref, out_ref):
        out_ref[...] = (dinv_ref[...] * (p_ref[0] + p_ref[1] + y_ref[...])
                        + b_ref[...])

    return pl.pallas_call(
        body,
        grid=(n // br,),
        in_specs=[
            pl.BlockSpec((NC, br, hw), lambda i: (0, i, 0)),
            pl.BlockSpec((br, hw), lambda i: (i, 0)),
            pl.BlockSpec((br, 1), lambda i: (i, 0)),
            pl.BlockSpec((1, hw), lambda i: (0, 0)),
        ],
        out_specs=pl.BlockSpec((br, hw), lambda i: (i, 0)),
        out_shape=jax.ShapeDtypeStruct((n, hw), jnp.float32),
    )


def kernel(x, edge_index, W1, b1, g1, be1, W2, b2, g2, be2, W3, b3):
    n, d = x.shape
    h = W1.shape[1]
    c_out = W3.shape[1]
    e = edge_index.shape[1]
    epw = e // NW
    g = 80
    ng = epw // g

    src2 = edge_index[0].reshape(NW, epw)
    dst3 = edge_index[1].reshape(NW, ng, g)

    br = 1000
    ra = (((n + NS - 1) // NS + 7) // 8) * 8
    zeros_h = jnp.zeros((ra, h), jnp.float32)
    ones_h = jnp.ones((g, h), jnp.float32)

    cnt = _make_count(n, h, ng, g)(dst3, ones_h, zeros_h)
    deg2 = cnt[:, :, 0]
    dinv_col = _make_dinv(n)(deg2).reshape(n, 1)

    y1 = _make_lin(n, d, h, br)(x, W1, dinv_col)
    p1 = _make_scatter(n, h, ng, g)(y1, src2, dst3, zeros_h)
    z1, s11, s12 = _make_post(n, h, br)(p1, y1, dinv_col, b1.reshape(1, h))
    y2 = _make_bnmm(n, h, h, br)(z1, s11, s12, g1.reshape(1, h),
                                 be1.reshape(1, h), W2, dinv_col)
    p2 = _make_scatter(n, h, ng, g)(y2, src2, dst3, zeros_h)
    z2, s21, s22 = _make_post(n, h, br)(p2, y2, dinv_col, b2.reshape(1, h))

    cp = h
    W3p = jnp.pad(W3, ((0, 0), (0, cp - c_out)))
    b3p = jnp.pad(b3, (0, cp - c_out)).reshape(1, cp)
    y3 = _make_bnmm(n, h, cp, br)(z2, s21, s22, g2.reshape(1, h),
                                  be2.reshape(1, h), W3p, dinv_col)
    zeros_c = jnp.zeros((ra, cp), jnp.float32)
    p3 = _make_scatter(n, cp, ng, g)(y3, src2, dst3, zeros_c)
    out = _make_comb(n, cp, br)(p3, y3, dinv_col, b3p)
    return out[:, :c_out]


# speedup vs baseline: 22.0932x; 1.0016x over previous
import functools

import jax
import jax.numpy as jnp
from jax import lax
from jax.experimental import pallas as pl
from jax.experimental.pallas import tpu as pltpu
import jax.experimental.pallas.tpu_sc as plsc

NC = 2
NS = 16
NW = NC * NS
LANES = 16


def _mesh():
    return plsc.VectorSubcoreMesh(
        core_axis_name="c", subcore_axis_name="s",
        num_cores=NC, num_subcores=NS)


_SC_PARAMS = pltpu.CompilerParams(needs_layout_passes=False)


@functools.cache
def _make_count(n, hw, ng, g):
    ra = (((n + NS - 1) // NS + 7) // 8) * 8
    rb = n - (NS - 1) * ra

    def body(dst_hbm, ones_hbm, zero_hbm, out_hbm, dst_v, ones_v, acc_sh,
             sem):
        c = lax.axis_index("c")
        s = lax.axis_index("s")
        wid = c * NS + s
        pltpu.sync_copy(dst_hbm.at[wid], dst_v)
        pltpu.sync_copy(ones_hbm, ones_v)
        base = s * ra

        @pl.when(s < NS - 1)
        def _():
            pltpu.sync_copy(zero_hbm, acc_sh.at[pl.ds(base, ra)])

        @pl.when(s == NS - 1)
        def _():
            pltpu.sync_copy(zero_hbm.at[pl.ds(0, rb)],
                            acc_sh.at[pl.ds((NS - 1) * ra, rb)])
        plsc.subcore_barrier()

        LAG = 4

        def grp(j, _):
            pltpu.async_copy(ones_v, acc_sh.at[dst_v.at[j]], sem, add=True)

            @pl.when(j >= LAG)
            def _():
                pltpu.make_async_copy(ones_v, acc_sh.at[dst_v.at[j - LAG]],
                                      sem).wait()
            return 0
        lax.fori_loop(0, ng, grp, 0)

        def drain(j, _):
            pltpu.make_async_copy(ones_v, acc_sh.at[dst_v.at[j]], sem).wait()
            return 0
        lax.fori_loop(ng - LAG, ng, drain, 0)

        plsc.subcore_barrier()

        @pl.when(s < NS - 1)
        def _():
            pltpu.sync_copy(acc_sh.at[pl.ds(base, ra)],
                            out_hbm.at[c, pl.ds(base, ra)])

        @pl.when(s == NS - 1)
        def _():
            pltpu.sync_copy(acc_sh.at[pl.ds((NS - 1) * ra, rb)],
                            out_hbm.at[c, pl.ds((NS - 1) * ra, rb)])

    return pl.kernel(
        body,
        out_type=jax.ShapeDtypeStruct((NC, n, hw), jnp.float32),
        mesh=_mesh(),
        compiler_params=_SC_PARAMS,
        scratch_types=[
            pltpu.VMEM((ng, g), jnp.int32),
            pltpu.VMEM((g, hw), jnp.float32),
            pltpu.VMEM_SHARED((n, hw), jnp.float32),
            pltpu.SemaphoreType.DMA,
        ],
    )


@functools.cache
def _make_scatter(n, hw, ng, g):
    ra = (((n + NS - 1) // NS + 7) // 8) * 8
    rb = n - (NS - 1) * ra
    assert rb > 0 and rb % 8 == 0

    epw = ng * g

    def body(y_hbm, src_hbm, dst_hbm, zero_hbm, out_hbm,
             src_v, dst_v, rows_v, acc_sh, sem0, sem1):
        c = lax.axis_index("c")
        s = lax.axis_index("s")
        wid = c * NS + s
        pltpu.sync_copy(src_hbm.at[wid], src_v)
        pltpu.sync_copy(dst_hbm.at[wid], dst_v)
        rows0_v = rows_v.at[0]
        rows1_v = rows_v.at[1]

        base = s * ra

        @pl.when(s < NS - 1)
        def _():
            pltpu.sync_copy(zero_hbm, acc_sh.at[pl.ds(base, ra)])

        @pl.when(s == NS - 1)
        def _():
            pltpu.sync_copy(zero_hbm.at[pl.ds(0, rb)],
                            acc_sh.at[pl.ds((NS - 1) * ra, rb)])
        plsc.subcore_barrier()

        def src_at(j):
            return src_v.at[pl.ds(pl.multiple_of(j * g, 8), g)]

        pltpu.async_copy(y_hbm.at[src_at(0)], rows0_v, sem0)
        pltpu.async_copy(y_hbm.at[src_at(1)], rows1_v, sem1)

        def grp2(j2, _):
            for b, (buf, sem) in enumerate(((rows0_v, sem0),
                                            (rows1_v, sem1))):
                j = j2 * 2 + b
                pltpu.make_async_copy(y_hbm.at[src_at(j)], buf, sem).wait()
                pltpu.sync_copy(buf, acc_sh.at[dst_v.at[j]], add=True)

                @pl.when(j + 2 < ng)
                def _():
                    pltpu.async_copy(y_hbm.at[src_at(j + 2)], buf, sem)
            return 0
        lax.fori_loop(0, ng // 2, grp2, 0)

        if ng % 2:
            pltpu.make_async_copy(y_hbm.at[src_at(ng - 1)], rows0_v,
                                  sem0).wait()
            pltpu.sync_copy(rows0_v, acc_sh.at[dst_v.at[ng - 1]], add=True)

        plsc.subcore_barrier()

        @pl.when(s < NS - 1)
        def _():
            pltpu.sync_copy(acc_sh.at[pl.ds(base, ra)],
                            out_hbm.at[c, pl.ds(base, ra)])

        @pl.when(s == NS - 1)
        def _():
            pltpu.sync_copy(acc_sh.at[pl.ds((NS - 1) * ra, rb)],
                            out_hbm.at[c, pl.ds((NS - 1) * ra, rb)])

    return pl.kernel(
        body,
        out_type=jax.ShapeDtypeStruct((NC, n, hw), jnp.float32),
        mesh=_mesh(),
        compiler_params=_SC_PARAMS,
        scratch_types=[
            pltpu.VMEM((epw,), jnp.int32),
            pltpu.VMEM((ng, g), jnp.int32),
            pltpu.VMEM((2, g, hw), jnp.float32),
            pltpu.VMEM_SHARED((n, hw), jnp.float32),
            pltpu.SemaphoreType.DMA,
            pltpu.SemaphoreType.DMA,
        ],
    )


@functools.cache
def _make_dinv(n):
    def body(deg_ref, out_ref):
        d = deg_ref[0:1, :] + deg_ref[1:2, :] + 1.0
        out_ref[...] = lax.rsqrt(d)

    return pl.pallas_call(
        body, out_shape=jax.ShapeDtypeStruct((1, n), jnp.float32))


@functools.cache
def _make_lin(n, d, h, br):
    def body(x_ref, w_ref, dinv_ref, out_ref):
        xw = jnp.dot(x_ref[...], w_ref[...],
                     preferred_element_type=jnp.float32)
        out_ref[...] = xw * dinv_ref[...]

    return pl.pallas_call(
        body,
        grid=(n // br,),
        in_specs=[
            pl.BlockSpec((br, d), lambda i: (i, 0)),
            pl.BlockSpec((d, h), lambda i: (0, 0)),
            pl.BlockSpec((br, 1), lambda i: (i, 0)),
        ],
        out_specs=pl.BlockSpec((br, h), lambda i: (i, 0)),
        out_shape=jax.ShapeDtypeStruct((n, h), jnp.float32),
    )


@functools.cache
def _make_post(n, hw, br):
    def body(p_ref, y_ref, dinv_ref, b_ref, z_ref, s1_ref, s2_ref):
        i = pl.program_id(0)
        z = dinv_ref[...] * (p_ref[0] + p_ref[1] + y_ref[...]) + b_ref[...]
        z_ref[...] = z

        @pl.when(i == 0)
        def _():
            s1_ref[...] = jnp.zeros_like(s1_ref)
            s2_ref[...] = jnp.zeros_like(s2_ref)
        s1_ref[...] += jnp.sum(z, axis=0, keepdims=True)
        s2_ref[...] += jnp.sum(z * z, axis=0, keepdims=True)

    return pl.pallas_call(
        body,
        grid=(n // br,),
        in_specs=[
            pl.BlockSpec((NC, br, hw), lambda i: (0, i, 0)),
            pl.BlockSpec((br, hw), lambda i: (i, 0)),
            pl.BlockSpec((br, 1), lambda i: (i, 0)),
            pl.BlockSpec((1, hw), lambda i: (0, 0)),
        ],
        out_specs=[
            pl.BlockSpec((br, hw), lambda i: (i, 0)),
            pl.BlockSpec((1, hw), lambda i: (0, 0)),
            pl.BlockSpec((1, hw), lambda i: (0, 0)),
        ],
        out_shape=[
            jax.ShapeDtypeStruct((n, hw), jnp.float32),
            jax.ShapeDtypeStruct((1, hw), jnp.float32),
            jax.ShapeDtypeStruct((1, hw), jnp.float32),
        ],
    )


@functools.cache
def _make_bnmm(n, hw, hout, br):
    inv_n = 1.0 / n

    def body(z_ref, s1_ref, s2_ref, g_ref, be_ref, w_ref, dinv_ref, out_ref):
        m = s1_ref[...] * inv_n
        var = s2_ref[...] * inv_n - m * m
        istd = lax.rsqrt(var + 1e-5)
        h = jnp.maximum((z_ref[...] - m) * (istd * g_ref[...]) + be_ref[...],
                        0.0)
        out_ref[...] = jnp.dot(h, w_ref[...],
                               preferred_element_type=jnp.float32) * dinv_ref[...]

    return pl.pallas_call(
        body,
        grid=(n // br,),
        in_specs=[
            pl.BlockSpec((br, hw), lambda i: (i, 0)),
            pl.BlockSpec((1, hw), lambda i: (0, 0)),
            pl.BlockSpec((1, hw), lambda i: (0, 0)),
            pl.BlockSpec((1, hw), lambda i: (0, 0)),
            pl.BlockSpec((1, hw), lambda i: (0, 0)),
            pl.BlockSpec((hw, hout), lambda i: (0, 0)),
            pl.BlockSpec((br, 1), lambda i: (i, 0)),
        ],
        out_specs=pl.BlockSpec((br, hout), lambda i: (i, 0)),
        out_shape=jax.ShapeDtypeStruct((n, hout), jnp.float32),
    )


@functools.cache
def _make_comb(n, hw, br):
    def body(p_ref, y_ref, dinv_ref, b_ref, out_ref):
        out_ref[...] = (dinv_ref[...] * (p_ref[0] + p_ref[1] + y_ref[...])
                        + b_ref[...])

    return pl.pallas_call(
        body,
        grid=(n // br,),
        in_specs=[
            pl.BlockSpec((NC, br, hw), lambda i: (0, i, 0)),
            pl.BlockSpec((br, hw), lambda i: (i, 0)),
            pl.BlockSpec((br, 1), lambda i: (i, 0)),
            pl.BlockSpec((1, hw), lambda i: (0, 0)),
        ],
        out_specs=pl.BlockSpec((br, hw), lambda i: (i, 0)),
        out_shape=jax.ShapeDtypeStruct((n, hw), jnp.float32),
    )


def kernel(x, edge_index, W1, b1, g1, be1, W2, b2, g2, be2, W3, b3):
    n, d = x.shape
    h = W1.shape[1]
    c_out = W3.shape[1]
    e = edge_index.shape[1]
    epw = e // NW
    g = 80
    ng = epw // g

    src2 = edge_index[0].reshape(NW, epw)
    dst3 = edge_index[1].reshape(NW, ng, g)

    br = 1000
    ra = (((n + NS - 1) // NS + 7) // 8) * 8
    zeros_h = jnp.zeros((ra, h), jnp.float32)
    ones_h = jnp.ones((g, h), jnp.float32)

    cnt = _make_count(n, h, ng, g)(dst3, ones_h, zeros_h)
    deg2 = cnt[:, :, 0]
    dinv_col = _make_dinv(n)(deg2).reshape(n, 1)

    y1 = _make_lin(n, d, h, br)(x, W1, dinv_col)
    p1 = _make_scatter(n, h, ng, g)(y1, src2, dst3, zeros_h)
    z1, s11, s12 = _make_post(n, h, br)(p1, y1, dinv_col, b1.reshape(1, h))
    y2 = _make_bnmm(n, h, h, br)(z1, s11, s12, g1.reshape(1, h),
                                 be1.reshape(1, h), W2, dinv_col)
    p2 = _make_scatter(n, h, ng, g)(y2, src2, dst3, zeros_h)
    z2, s21, s22 = _make_post(n, h, br)(p2, y2, dinv_col, b2.reshape(1, h))

    cp = h
    W3p = jnp.pad(W3, ((0, 0), (0, cp - c_out)))
    b3p = jnp.pad(b3, (0, cp - c_out)).reshape(1, cp)
    y3 = _make_bnmm(n, h, cp, br)(z2, s21, s22, g2.reshape(1, h),
                                  be2.reshape(1, h), W3p, dinv_col)
    zeros_c = jnp.zeros((ra, cp), jnp.float32)
    p3 = _make_scatter(n, cp, ng, g)(y3, src2, dst3, zeros_c)
    out = _make_comb(n, cp, br)(p3, y3, dinv_col, b3p)
    return out[:, :c_out]
